# baseline (speedup 1.0000x reference)
import jax
import jax.numpy as jnp
from jax.experimental import pallas as pl
from jax.experimental.pallas import tpu as pltpu

_MASK_TOKEN_ID = -10.0
_PAD_TOKEN_ID = -20.0
_NEG_SLOPE = 0.1
_BIN_ALPHA = 1.0
_PBIN = 128
_BLOCK = 2048


def _body(x_ref, w1_ref, b1_ref, w2_ref, b2_ref, emb_ref, em_ref, ep_ref,
          o_ref):
    x = x_ref[...]
    h = x * w1_ref[...] + b1_ref[...]
    h = jnp.where(h >= 0, h, _NEG_SLOPE * h)
    h2 = jnp.dot(h, w2_ref[...],
                 preferred_element_type=jnp.float32) + b2_ref[...]
    m = jnp.max(h2, axis=-1, keepdims=True)
    e = jnp.exp(h2 - m)
    s = jnp.sum(e, axis=-1, keepdims=True)
    out = jnp.dot(e, emb_ref[...],
                  preferred_element_type=jnp.float32) / s
    out = jnp.where(x == _MASK_TOKEN_ID, em_ref[...], out)
    out = jnp.where(x == _PAD_TOKEN_ID, ep_ref[...], out)
    o_ref[...] = out


def kernel(x, W1, b1, W2, b2, emb, emb_mask, emb_pad):
    B, L, _ = x.shape
    nbin = W1.shape[1]
    dim = emb.shape[1]
    T = B * L
    f32 = jnp.float32

    w1p = jnp.zeros((1, _PBIN), f32).at[:, :nbin].set(W1.astype(f32))
    b1p = jnp.zeros((1, _PBIN), f32).at[:, :nbin].set(b1.astype(f32))
    w2p = jnp.zeros((_PBIN, _PBIN), f32).at[:nbin, :nbin].set(
        _BIN_ALPHA * jnp.eye(nbin, dtype=f32) + W2.astype(f32))
    b2p = jnp.full((1, _PBIN), -1e30, f32).at[:, :nbin].set(b2.astype(f32))
    embp = jnp.zeros((_PBIN, dim), f32).at[:nbin].set(emb.astype(f32))

    xf = x.reshape(T, 1)
    grid = T // _BLOCK

    full = lambda shape: pl.BlockSpec(shape, lambda i: (0, 0))
    out = pl.pallas_call(
        _body,
        grid=(grid,),
        in_specs=[
            pl.BlockSpec((_BLOCK, 1), lambda i: (i, 0)),
            full((1, _PBIN)),
            full((1, _PBIN)),
            full((_PBIN, _PBIN)),
            full((1, _PBIN)),
            full((_PBIN, dim)),
            full((1, dim)),
            full((1, dim)),
        ],
        out_specs=pl.BlockSpec((_BLOCK, dim), lambda i: (i, 0)),
        out_shape=jax.ShapeDtypeStruct((T, dim), f32),
        compiler_params=pltpu.CompilerParams(
            dimension_semantics=("arbitrary",)),
    )(xf, w1p, b1p, w2p, b2p, embp,
      emb_mask.astype(f32), emb_pad.astype(f32))
    return out.reshape(B, L, dim)
